# Initial kernel scaffold; baseline (speedup 1.0000x reference)
import jax
import jax.numpy as jnp
from jax.experimental import pallas as pl


def kernel(macro_features, edge_index, Wa, ba, Wp, bp, Wzm, bzm, Wg, bg, Wm, bm):
    raise NotImplementedError("write your pallas kernel here")


# speedup vs baseline: 1.6191x; 1.6191x over previous
import functools
import jax
import jax.numpy as jnp
from jax import lax
from jax.experimental import pallas as pl
from jax.experimental.pallas import tpu as pltpu

N = 10000
E = 160000
D = 128
Q = 128
MAP = 64
H = 2

NB = 1000
GRID = N // NB


def _pre_body(x_ref, wnt_ref, bnt_ref, wzm_ref, bzm_ref, wp0_ref, bp0_ref,
              wp1_ref, bp1_ref, nt_ref, z_ref, p0_ref, p1_ref):
    x = x_ref[...]
    nt_ref[...] = jnp.dot(x, wnt_ref[...],
                          preferred_element_type=jnp.float32) + bnt_ref[...]
    z_ref[...] = jnp.dot(x, wzm_ref[...],
                         preferred_element_type=jnp.float32) + bzm_ref[...]
    p0_ref[...] = jnp.dot(x, wp0_ref[...],
                          preferred_element_type=jnp.float32) + bp0_ref[...]
    p1_ref[...] = jnp.dot(x, wp1_ref[...],
                          preferred_element_type=jnp.float32) + bp1_ref[...]


def _pre(x, Wnt, bnt, Wzm, bzm, Wp0, bp0, Wp1, bp1):
    full = lambda s: pl.BlockSpec(s, lambda i: (0, 0))
    row = lambda w: pl.BlockSpec((NB, w), lambda i: (i, 0))
    return pl.pallas_call(
        _pre_body,
        grid=(GRID,),
        in_specs=[row(D), full((D, 8)), pl.BlockSpec((1, 8), lambda i: (0, 0)),
                  full((D, MAP)), pl.BlockSpec((1, MAP), lambda i: (0, 0)),
                  full((D, Q)), pl.BlockSpec((1, Q), lambda i: (0, 0)),
                  full((D, Q)), pl.BlockSpec((1, Q), lambda i: (0, 0))],
        out_specs=[row(8), row(MAP), row(Q), row(Q)],
        out_shape=[jax.ShapeDtypeStruct((N, 8), jnp.float32),
                   jax.ShapeDtypeStruct((N, MAP), jnp.float32),
                   jax.ShapeDtypeStruct((N, Q), jnp.float32),
                   jax.ShapeDtypeStruct((N, Q), jnp.float32)],
    )(x, Wnt, bnt.reshape(1, 8), Wzm, bzm.reshape(1, MAP),
      Wp0, bp0.reshape(1, Q), Wp1, bp1.reshape(1, Q))


def _post_body(x_ref, zmax_ref, pack_ref, h0_ref, h1_ref, wgx_ref, wgz_ref,
               bg_ref, wm0_ref, wm1_ref, wm2_ref, bm_ref, out_ref):
    x = x_ref[...]
    pack = pack_ref[...]
    deg = pack[:, 4:5]
    has_edge = deg > 0.0
    zmax = jnp.where(has_edge, zmax_ref[...], 0.0)
    usum = pack[:, 2:4]
    xmean_term = usum / jnp.maximum(deg, 1.0)
    glogit = (jnp.dot(x, wgx_ref[...], preferred_element_type=jnp.float32)
              + jnp.dot(zmax, wgz_ref[...], preferred_element_type=jnp.float32)
              + xmean_term + bg_ref[...])
    gates = jax.nn.sigmoid(glogit)
    denom = pack[:, 0:2] + 1e-16
    m0 = gates[:, 0:1] * h0_ref[...] / denom[:, 0:1]
    m1 = gates[:, 1:2] * h1_ref[...] / denom[:, 1:2]
    out = (jnp.dot(x, wm0_ref[...], preferred_element_type=jnp.float32)
           + jnp.dot(m0, wm1_ref[...], preferred_element_type=jnp.float32)
           + jnp.dot(m1, wm2_ref[...], preferred_element_type=jnp.float32)
           + bm_ref[...])
    out = jnp.maximum(out, 0.0)
    vel = x[:, 0:1]
    zero_mask = jnp.abs(vel) < 0.5
    max_mask = jnp.abs(vel - 70.0) < 0.5
    learning = jnp.logical_not(jnp.logical_or(zero_mask, max_mask))
    lastcol = (lax.broadcasted_iota(jnp.int32, out.shape, 1) == (Q - 1))
    base = jnp.where(jnp.logical_and(max_mask, lastcol),
                     (Q / 70.0) * vel, 0.0)
    out_ref[...] = jnp.where(learning, out, base)


def _post(x, zmax, pack, h0, h1, Wgx, Wgz, bg, Wm0, Wm1, Wm2, bm):
    full = lambda s: pl.BlockSpec(s, lambda i: (0, 0))
    row = lambda w: pl.BlockSpec((NB, w), lambda i: (i, 0))
    return pl.pallas_call(
        _post_body,
        grid=(GRID,),
        in_specs=[row(D), row(MAP), row(16), row(Q), row(Q),
                  full((D, H)), full((MAP, H)),
                  pl.BlockSpec((1, H), lambda i: (0, 0)),
                  full((D, Q)), full((Q, Q)), full((Q, Q)),
                  pl.BlockSpec((1, Q), lambda i: (0, 0))],
        out_specs=row(Q),
        out_shape=jax.ShapeDtypeStruct((N, Q), jnp.float32),
    )(x, zmax, pack, h0, h1, Wgx, Wgz, bg.reshape(1, H),
      Wm0, Wm1, Wm2, bm.reshape(1, Q))


def kernel(macro_features, edge_index, Wa, ba, Wp, bp, Wzm, bzm, Wg, bg, Wm, bm):
    x = macro_features
    src = edge_index[0]
    dst = edge_index[1]

    Wnt = jnp.concatenate([Wa[:D], Wa[D:], Wg[D + MAP:], jnp.zeros((D, 2))], axis=1)
    bnt = jnp.concatenate([ba, jnp.zeros((6,))])
    nt, z, p0, p1 = _pre(x, Wnt, bnt, Wzm, bzm,
                         Wp[:, :Q], bp[:Q], Wp[:, Q:], bp[Q:])

    e = nt[dst, 0:2] + nt[src, 2:4]
    e = jnp.where(e > 0, e, 0.2 * e)
    w = jnp.exp(e)
    denom = jax.ops.segment_sum(w, dst, num_segments=N)
    usum = jax.ops.segment_sum(nt[src, 4:6], dst, num_segments=N)
    deg = jax.ops.segment_sum(jnp.ones((E,), jnp.float32), dst, num_segments=N)
    zmax = jax.ops.segment_max(z[src], dst, num_segments=N)
    zmax = jnp.where(jnp.isfinite(zmax), zmax, 0.0)
    h0 = jax.ops.segment_sum(w[:, 0:1] * p0[src], dst, num_segments=N)
    h1 = jax.ops.segment_sum(w[:, 1:2] * p1[src], dst, num_segments=N)
    pack = jnp.concatenate(
        [denom, usum, deg[:, None], jnp.zeros((N, 11), jnp.float32)], axis=1)

    return _post(x, zmax, pack, h0, h1,
                 Wg[:D], Wg[D:D + MAP], bg,
                 Wm[:D], Wm[D:D + Q], Wm[D + Q:], bm)
